# baseline (speedup 1.0000x reference)
import functools
import math

import jax
import jax.numpy as jnp
from jax import lax
from jax.experimental import pallas as pl
from jax.experimental.pallas import tpu as pltpu
from jax.experimental.pallas import tpu_sc as plsc

N = 10000
D_IN = 128
D_H = 256
D_LAT = 128
BN_EPS = 1e-5

E_RAW = 320000
E_FULL = E_RAW + N
NTILES = 16
E_PAD = 331776
PER_TILE = E_PAD // NTILES

CHUNK = 128
DEG_NPAD = 10112
DEG_CPT = 81
DEG_PER_TILE = DEG_CPT * CHUNK
DEG_ZROWS = DEG_NPAD // NTILES

NPAD = 10008
CH = 128
CPT = PER_TILE // CH
NR = 3
NI = 6
ZR = 632
ZR_LAST = NPAD - 15 * ZR
assert CPT % 6 == 0

_mesh = plsc.VectorSubcoreMesh(core_axis_name="c", subcore_axis_name="s")


@functools.partial(
    pl.kernel,
    out_type=jax.ShapeDtypeStruct((2 * DEG_NPAD, 16), jnp.float32),
    mesh=_mesh,
    scratch_types=[
        pltpu.VMEM_SHARED((DEG_NPAD, 16), jnp.float32),
        pltpu.VMEM((DEG_ZROWS, 16), jnp.float32),
        pltpu.VMEM((CHUNK, 16), jnp.float32),
        pltpu.VMEM((CHUNK,), jnp.int32),
    ],
)
def _deg_kernel(dst_hbm, out_hbm, acc, zbuf, ones_v, idxd):
    c = lax.axis_index("c")
    s = lax.axis_index("s")

    def _fill(j, _):
        zbuf[j, :] = jnp.zeros((16,), jnp.float32)
        return 0

    lax.fori_loop(0, DEG_ZROWS, _fill, 0)

    def _fill1(j, _):
        ones_v[j, :] = jnp.ones((16,), jnp.float32)
        return 0

    lax.fori_loop(0, CHUNK, _fill1, 0)
    pltpu.sync_copy(zbuf, acc.at[pl.ds(s * DEG_ZROWS, DEG_ZROWS)])
    plsc.subcore_barrier()

    base = c * (E_PAD // 2) + s * DEG_PER_TILE

    def _body(g, _):
        pltpu.sync_copy(dst_hbm.at[pl.ds(base + g * CHUNK, CHUNK)], idxd)
        pltpu.sync_copy(ones_v, acc.at[idxd], add=True)
        return 0

    lax.fori_loop(0, DEG_CPT, _body, 0)
    plsc.subcore_barrier()
    pltpu.sync_copy(acc.at[pl.ds(s * DEG_ZROWS, DEG_ZROWS)],
                    out_hbm.at[pl.ds(c * DEG_NPAD + s * DEG_ZROWS, DEG_ZROWS)])


_AGG_SCRATCH = (
    [pltpu.VMEM_SHARED((NPAD, 128), jnp.float32)]
    + [pltpu.VMEM((CH, 128), jnp.float32) for _ in range(NR)]
    + [pltpu.VMEM((CH,), jnp.int32) for _ in range(NI)]
    + [pltpu.VMEM((CH,), jnp.int32) for _ in range(NI)]
    + [pltpu.SemaphoreType.DMA for _ in range(NR)]
    + [pltpu.SemaphoreType.DMA for _ in range(NR)]
    + [pltpu.SemaphoreType.DMA for _ in range(NI)]
)


@functools.partial(
    pl.kernel,
    out_type=jax.ShapeDtypeStruct((2 * NPAD, 128), jnp.float32),
    mesh=_mesh,
    scratch_types=_AGG_SCRATCH,
)
def _agg_kernel(tab_hbm, srcs_hbm, dsts_hbm, out_hbm, acc, *scr):
    rows = scr[0:NR]
    isb = scr[NR:NR + NI]
    idb = scr[NR + NI:NR + 2 * NI]
    gsem = scr[NR + 2 * NI:2 * NR + 2 * NI]
    ssem = scr[2 * NR + 2 * NI:3 * NR + 2 * NI]
    isem = scr[3 * NR + 2 * NI:3 * NR + 3 * NI]
    c = lax.axis_index("c")
    s = lax.axis_index("s")

    def _fill(j, _):
        for k in range(8):
            rows[0][j, pl.ds(k * 16, 16)] = jnp.zeros((16,), jnp.float32)
        return 0

    lax.fori_loop(0, CH, _fill, 0)
    nz_full = ZR // CH
    base = s * ZR

    @pl.when(s < 15)
    def _zero_main():
        for t in range(nz_full):
            pltpu.sync_copy(rows[0], acc.at[pl.ds(base + t * CH, CH)])
        pltpu.sync_copy(rows[0].at[pl.ds(0, ZR - nz_full * CH)],
                        acc.at[pl.ds(base + nz_full * CH, ZR - nz_full * CH)])

    @pl.when(s == 15)
    def _zero_last():
        for t in range(ZR_LAST // CH):
            pltpu.sync_copy(rows[0], acc.at[pl.ds(base + t * CH, CH)])
        rem = ZR_LAST % CH
        pltpu.sync_copy(rows[0].at[pl.ds(0, rem)],
                        acc.at[pl.ds(base + (ZR_LAST // CH) * CH, rem)])

    plsc.subcore_barrier()

    ebase = s * PER_TILE
    sbase = c * E_PAD + ebase

    def _idx_start(g, i):
        pltpu.async_copy(srcs_hbm.at[pl.ds(sbase + g * CH, CH)], isb[i], isem[i])
        pltpu.async_copy(dsts_hbm.at[pl.ds(ebase + g * CH, CH)], idb[i], isem[i])

    def _idx_wait(g, i):
        pltpu.make_async_copy(srcs_hbm.at[pl.ds(sbase + g * CH, CH)], isb[i],
                              isem[i]).wait()
        pltpu.make_async_copy(dsts_hbm.at[pl.ds(ebase + g * CH, CH)], idb[i],
                              isem[i]).wait()

    for u in range(4):
        _idx_start(u, u)
    for u in range(2):
        _idx_wait(u, u)
        pltpu.async_copy(tab_hbm.at[isb[u]], rows[u], gsem[u])

    def _body(j, _):
        for u in range(6):
            g = 6 * j + u
            r = u % NR
            i = u % NI
            pltpu.make_async_copy(tab_hbm.at[isb[i]], rows[r], gsem[r]).wait()
            pltpu.async_copy(rows[r], acc.at[idb[i]], ssem[r], add=True)

            @pl.when(g >= 1)
            def _wait_sm1():
                pltpu.make_async_copy(rows[(u + 2) % NR],
                                      acc.at[idb[(u + 5) % NI]],
                                      ssem[(u + 2) % NR]).wait()

            @pl.when(g + 4 < CPT)
            def _pf_idx():
                _idx_start(g + 4, (u + 4) % NI)

            @pl.when(g + 2 < CPT)
            def _next_gather():
                _idx_wait(g + 2, (u + 2) % NI)
                pltpu.async_copy(tab_hbm.at[isb[(u + 2) % NI]],
                                 rows[(u + 2) % NR], gsem[(u + 2) % NR])

        return 0

    lax.fori_loop(0, CPT // 6, _body, 0)
    g = CPT - 1
    pltpu.make_async_copy(rows[g % NR], acc.at[idb[g % NI]],
                          ssem[g % NR]).wait()
    plsc.subcore_barrier()

    @pl.when(s < 15)
    def _out_main():
        pltpu.sync_copy(acc.at[pl.ds(s * ZR, ZR)],
                        out_hbm.at[pl.ds(c * NPAD + s * ZR, ZR)])

    @pl.when(s == 15)
    def _out_last():
        pltpu.sync_copy(acc.at[pl.ds(s * ZR, ZR_LAST)],
                        out_hbm.at[pl.ds(c * NPAD + s * ZR, ZR_LAST)])


def _mm1_body(x_ref, w_ref, p_ref, o_ref):
    deg = p_ref[0, :, 0:1] + p_ref[1, :, 0:1]
    dinv = lax.rsqrt(deg)
    o_ref[...] = jnp.dot(x_ref[...], w_ref[...],
                         preferred_element_type=jnp.float32) * dinv


def _mm2_body(al_ref, ah_ref, w_ref, p_ref, s_ref, b_ref, o_ref):
    deg = p_ref[0, :, 0:1] + p_ref[1, :, 0:1]
    dinv = lax.rsqrt(deg)
    h = jnp.concatenate([al_ref[...], ah_ref[...]], axis=1) * dinv
    h = jnp.maximum(h * s_ref[...] + b_ref[...], 0.0)
    o_ref[...] = jnp.dot(h, w_ref[...],
                         preferred_element_type=jnp.float32) * dinv


def _mm3_body(al_ref, ah_ref, w_ref, p_ref, s_ref, b_ref, bp_ref, o_ref):
    deg = p_ref[0, :, 0:1] + p_ref[1, :, 0:1]
    dinv = lax.rsqrt(deg)
    h = jnp.concatenate([al_ref[...], ah_ref[...]], axis=1) * dinv
    h = jnp.maximum(h * s_ref[...] + b_ref[...], 0.0)
    o_ref[...] = jnp.dot(h, w_ref[...],
                         preferred_element_type=jnp.float32) + bp_ref[...]


_RB = 1000


def kernel(x, edge_index, batch, W1, b1, g1, bt1, W2, b2, g2, bt2, Wp, bp):
    f32 = jnp.float32
    src = edge_index[0].astype(jnp.int32)
    dst = edge_index[1].astype(jnp.int32)
    loop = jnp.arange(N, dtype=jnp.int32)
    pad = E_PAD - E_FULL
    src_f = jnp.concatenate([src, loop, jnp.zeros((pad,), jnp.int32)])
    dst_f = jnp.concatenate([dst, loop, jnp.full((pad,), N, jnp.int32)])
    srcs2 = jnp.concatenate([2 * src_f, 2 * src_f + 1])

    sc = 1.0 / math.sqrt(1.0 + BN_EPS)
    s1 = (g1 * sc).reshape(1, D_H)
    c1 = (b1 * (g1 * sc) + bt1).reshape(1, D_H)
    s2 = (g2 * sc).reshape(1, D_H)
    c2 = (b2 * (g2 * sc) + bt2).reshape(1, D_H)
    bp2 = bp.reshape(1, D_LAT)

    degp = _deg_kernel(dst_f).reshape(2, DEG_NPAD, 16)

    t1 = pl.pallas_call(
        _mm1_body,
        grid=(N // _RB,),
        in_specs=[
            pl.BlockSpec((_RB, D_IN), lambda i: (i, 0)),
            pl.BlockSpec((D_IN, D_H), lambda i: (0, 0)),
            pl.BlockSpec((2, _RB, 16), lambda i: (0, i, 0)),
        ],
        out_specs=pl.BlockSpec((_RB, D_H), lambda i: (i, 0)),
        out_shape=jax.ShapeDtypeStruct((N, D_H), f32),
    )(x, W1, degp)

    a1 = _agg_kernel(t1.reshape(2 * N, 128), srcs2, dst_f)

    t2 = pl.pallas_call(
        _mm2_body,
        grid=(N // _RB,),
        in_specs=[
            pl.BlockSpec((_RB, 128), lambda i: (i, 0)),
            pl.BlockSpec((_RB, 128), lambda i: (i, 0)),
            pl.BlockSpec((D_H, D_H), lambda i: (0, 0)),
            pl.BlockSpec((2, _RB, 16), lambda i: (0, i, 0)),
            pl.BlockSpec((1, D_H), lambda i: (0, 0)),
            pl.BlockSpec((1, D_H), lambda i: (0, 0)),
        ],
        out_specs=pl.BlockSpec((_RB, D_H), lambda i: (i, 0)),
        out_shape=jax.ShapeDtypeStruct((N, D_H), f32),
    )(a1[0:N], a1[NPAD:NPAD + N], W2, degp, s1, c1)

    a2 = _agg_kernel(t2.reshape(2 * N, 128), srcs2, dst_f)

    z = pl.pallas_call(
        _mm3_body,
        grid=(N // _RB,),
        in_specs=[
            pl.BlockSpec((_RB, 128), lambda i: (i, 0)),
            pl.BlockSpec((_RB, 128), lambda i: (i, 0)),
            pl.BlockSpec((D_H, D_LAT), lambda i: (0, 0)),
            pl.BlockSpec((2, _RB, 16), lambda i: (0, i, 0)),
            pl.BlockSpec((1, D_H), lambda i: (0, 0)),
            pl.BlockSpec((1, D_H), lambda i: (0, 0)),
            pl.BlockSpec((1, D_LAT), lambda i: (0, 0)),
        ],
        out_specs=pl.BlockSpec((_RB, D_LAT), lambda i: (i, 0)),
        out_shape=jax.ShapeDtypeStruct((N, D_LAT), f32),
    )(a2[0:N], a2[NPAD:NPAD + N], Wp, degp, s2, c2, bp2)

    return z


# speedup vs baseline: 1.0625x; 1.0625x over previous
import functools
import math

import jax
import jax.numpy as jnp
from jax import lax
from jax.experimental import pallas as pl
from jax.experimental.pallas import tpu as pltpu
from jax.experimental.pallas import tpu_sc as plsc

N = 10000
D_IN = 128
D_H = 256
D_LAT = 128
BN_EPS = 1e-5

E_RAW = 320000
E_FULL = E_RAW + N
NTILES = 16
E_PAD = 331776
PER_TILE = E_PAD // NTILES

DEG_CH = 96
DEG_NPAD = 10112
DEG_CPT = 108
DEG_PER_TILE = DEG_CPT * DEG_CH
DEG_ZROWS = DEG_NPAD // NTILES
DEG_NI = 4

NPAD = 10008
CH = 96
CPT = PER_TILE // CH
NR = 4
NI = 6
ZR = 632
ZR_LAST = NPAD - 15 * ZR

_mesh = plsc.VectorSubcoreMesh(core_axis_name="c", subcore_axis_name="s")


@functools.partial(
    pl.kernel,
    out_type=jax.ShapeDtypeStruct((2 * DEG_NPAD, 16), jnp.float32),
    mesh=_mesh,
    scratch_types=(
        [pltpu.VMEM_SHARED((DEG_NPAD, 16), jnp.float32),
         pltpu.VMEM((DEG_ZROWS, 16), jnp.float32),
         pltpu.VMEM((DEG_CH, 16), jnp.float32)]
        + [pltpu.VMEM((DEG_CH,), jnp.int32) for _ in range(DEG_NI)]
        + [pltpu.SemaphoreType.DMA for _ in range(2 * DEG_NI)]
    ),
)
def _deg_kernel(dst_hbm, out_hbm, acc, zbuf, ones_v, *scr):
    idb = scr[0:DEG_NI]
    isem = scr[DEG_NI:2 * DEG_NI]
    ssem = scr[2 * DEG_NI:3 * DEG_NI]
    c = lax.axis_index("c")
    s = lax.axis_index("s")

    def _fill(j, _):
        zbuf[j, :] = jnp.zeros((16,), jnp.float32)
        return 0

    lax.fori_loop(0, DEG_ZROWS, _fill, 0)

    def _fill1(j, _):
        ones_v[j, :] = jnp.ones((16,), jnp.float32)
        return 0

    lax.fori_loop(0, DEG_CH, _fill1, 0)
    pltpu.sync_copy(zbuf, acc.at[pl.ds(s * DEG_ZROWS, DEG_ZROWS)])
    plsc.subcore_barrier()

    base = c * (E_PAD // 2) + s * DEG_PER_TILE

    def _idx_start(g, i):
        pltpu.async_copy(dst_hbm.at[pl.ds(base + g * DEG_CH, DEG_CH)],
                         idb[i], isem[i])

    def _idx_wait(g, i):
        pltpu.make_async_copy(dst_hbm.at[pl.ds(base + g * DEG_CH, DEG_CH)],
                              idb[i], isem[i]).wait()

    for u in range(2):
        _idx_start(u, u)

    def _body(j, _):
        for u in range(DEG_NI):
            g = DEG_NI * j + u
            _idx_wait(g, u)
            pltpu.async_copy(ones_v, acc.at[idb[u]], ssem[u], add=True)

            @pl.when(g >= 2)
            def _wait_sm2():
                pltpu.make_async_copy(ones_v, acc.at[idb[(u + 2) % DEG_NI]],
                                      ssem[(u + 2) % DEG_NI]).wait()

            @pl.when(g + 2 < DEG_CPT)
            def _pf():
                _idx_start(g + 2, (u + 2) % DEG_NI)

        return 0

    lax.fori_loop(0, DEG_CPT // DEG_NI, _body, 0)
    for g in (DEG_CPT - 2, DEG_CPT - 1):
        pltpu.make_async_copy(ones_v, acc.at[idb[g % DEG_NI]],
                              ssem[g % DEG_NI]).wait()
    plsc.subcore_barrier()
    pltpu.sync_copy(acc.at[pl.ds(s * DEG_ZROWS, DEG_ZROWS)],
                    out_hbm.at[pl.ds(c * DEG_NPAD + s * DEG_ZROWS, DEG_ZROWS)])


_AGG_SCRATCH = (
    [pltpu.VMEM_SHARED((NPAD, 128), jnp.float32)]
    + [pltpu.VMEM((CH, 128), jnp.float32) for _ in range(NR)]
    + [pltpu.VMEM((CH,), jnp.int32) for _ in range(NI)]
    + [pltpu.VMEM((CH,), jnp.int32) for _ in range(NI)]
    + [pltpu.SemaphoreType.DMA for _ in range(NR)]
    + [pltpu.SemaphoreType.DMA for _ in range(NR)]
    + [pltpu.SemaphoreType.DMA for _ in range(NI)]
)


@functools.partial(
    pl.kernel,
    out_type=jax.ShapeDtypeStruct((2, NPAD, 128), jnp.float32),
    mesh=_mesh,
    scratch_types=_AGG_SCRATCH,
)
def _agg_kernel(tab_hbm, srcs_hbm, dsts_hbm, out_hbm, acc, *scr):
    rows = scr[0:NR]
    isb = scr[NR:NR + NI]
    idb = scr[NR + NI:NR + 2 * NI]
    gsem = scr[NR + 2 * NI:2 * NR + 2 * NI]
    ssem = scr[2 * NR + 2 * NI:3 * NR + 2 * NI]
    isem = scr[3 * NR + 2 * NI:3 * NR + 3 * NI]
    c = lax.axis_index("c")
    s = lax.axis_index("s")

    def _fill(j, _):
        for k in range(8):
            rows[0][j, pl.ds(k * 16, 16)] = jnp.zeros((16,), jnp.float32)
        return 0

    lax.fori_loop(0, CH, _fill, 0)
    nz_full = ZR // CH
    base = s * ZR

    @pl.when(s < 15)
    def _zero_main():
        for t in range(nz_full):
            pltpu.sync_copy(rows[0], acc.at[pl.ds(base + t * CH, CH)])
        pltpu.sync_copy(rows[0].at[pl.ds(0, ZR - nz_full * CH)],
                        acc.at[pl.ds(base + nz_full * CH, ZR - nz_full * CH)])

    @pl.when(s == 15)
    def _zero_last():
        for t in range(ZR_LAST // CH):
            pltpu.sync_copy(rows[0], acc.at[pl.ds(base + t * CH, CH)])
        rem = ZR_LAST % CH
        pltpu.sync_copy(rows[0].at[pl.ds(0, rem)],
                        acc.at[pl.ds(base + (ZR_LAST // CH) * CH, rem)])

    plsc.subcore_barrier()

    ebase = s * PER_TILE
    sbase = c * E_PAD + ebase

    def _idx_start(g, i):
        pltpu.async_copy(srcs_hbm.at[pl.ds(sbase + g * CH, CH)], isb[i], isem[i])
        pltpu.async_copy(dsts_hbm.at[pl.ds(ebase + g * CH, CH)], idb[i], isem[i])

    def _idx_wait(g, i):
        pltpu.make_async_copy(srcs_hbm.at[pl.ds(sbase + g * CH, CH)], isb[i],
                              isem[i]).wait()
        pltpu.make_async_copy(dsts_hbm.at[pl.ds(ebase + g * CH, CH)], idb[i],
                              isem[i]).wait()

    for u in range(5):
        _idx_start(u, u)
    for u in range(3):
        _idx_wait(u, u)
        pltpu.async_copy(tab_hbm.at[isb[u]], rows[u], gsem[u])

    def _body(j, _):
        for u in range(12):
            g = 12 * j + u
            r = u % NR
            i = u % NI
            pltpu.make_async_copy(tab_hbm.at[isb[i]], rows[r], gsem[r]).wait()
            pltpu.async_copy(rows[r], acc.at[idb[i]], ssem[r], add=True)

            @pl.when(g >= 1)
            def _wait_sm1():
                pltpu.make_async_copy(rows[(u + 3) % NR],
                                      acc.at[idb[(u + 5) % NI]],
                                      ssem[(u + 3) % NR]).wait()

            @pl.when(g + 5 < CPT)
            def _pf_idx():
                _idx_start(g + 5, (u + 5) % NI)

            @pl.when(g + 3 < CPT)
            def _next_gather():
                _idx_wait(g + 3, (u + 3) % NI)
                pltpu.async_copy(tab_hbm.at[isb[(u + 3) % NI]],
                                 rows[(u + 3) % NR], gsem[(u + 3) % NR])

        return 0

    lax.fori_loop(0, CPT // 12, _body, 0)
    g = CPT - 1
    pltpu.make_async_copy(rows[g % NR], acc.at[idb[g % NI]],
                          ssem[g % NR]).wait()
    plsc.subcore_barrier()

    @pl.when(s < 15)
    def _out_main():
        pltpu.sync_copy(acc.at[pl.ds(s * ZR, ZR)],
                        out_hbm.at[c, pl.ds(s * ZR, ZR)])

    @pl.when(s == 15)
    def _out_last():
        pltpu.sync_copy(acc.at[pl.ds(s * ZR, ZR_LAST)],
                        out_hbm.at[c, pl.ds(s * ZR, ZR_LAST)])


def _mm1_body(x_ref, w_ref, p_ref, o_ref):
    deg = p_ref[0, :, 0:1] + p_ref[1, :, 0:1]
    dinv = lax.rsqrt(deg)
    o_ref[...] = jnp.dot(x_ref[...], w_ref[...],
                         preferred_element_type=jnp.float32) * dinv


def _mm2_body(al_ref, ah_ref, w_ref, p_ref, s_ref, b_ref, o_ref):
    deg = p_ref[0, :, 0:1] + p_ref[1, :, 0:1]
    dinv = lax.rsqrt(deg)
    h = jnp.concatenate([al_ref[0], ah_ref[0]], axis=1) * dinv
    h = jnp.maximum(h * s_ref[...] + b_ref[...], 0.0)
    o_ref[...] = jnp.dot(h, w_ref[...],
                         preferred_element_type=jnp.float32) * dinv


def _mm3_body(al_ref, ah_ref, w_ref, p_ref, s_ref, b_ref, bp_ref, o_ref):
    deg = p_ref[0, :, 0:1] + p_ref[1, :, 0:1]
    dinv = lax.rsqrt(deg)
    h = jnp.concatenate([al_ref[0], ah_ref[0]], axis=1) * dinv
    h = jnp.maximum(h * s_ref[...] + b_ref[...], 0.0)
    o_ref[...] = jnp.dot(h, w_ref[...],
                         preferred_element_type=jnp.float32) + bp_ref[...]


_RB = 1000


def kernel(x, edge_index, batch, W1, b1, g1, bt1, W2, b2, g2, bt2, Wp, bp):
    f32 = jnp.float32
    src = edge_index[0].astype(jnp.int32)
    dst = edge_index[1].astype(jnp.int32)
    loop = jnp.arange(N, dtype=jnp.int32)
    pad = E_PAD - E_FULL
    src_f = jnp.concatenate([src, loop, jnp.zeros((pad,), jnp.int32)])
    dst_f = jnp.concatenate([dst, loop, jnp.full((pad,), N, jnp.int32)])
    srcs2 = jnp.concatenate([2 * src_f, 2 * src_f + 1])

    sc = 1.0 / math.sqrt(1.0 + BN_EPS)
    s1 = (g1 * sc).reshape(1, D_H)
    c1 = (b1 * (g1 * sc) + bt1).reshape(1, D_H)
    s2 = (g2 * sc).reshape(1, D_H)
    c2 = (b2 * (g2 * sc) + bt2).reshape(1, D_H)
    bp2 = bp.reshape(1, D_LAT)

    degp = _deg_kernel(dst_f).reshape(2, DEG_NPAD, 16)

    t1 = pl.pallas_call(
        _mm1_body,
        grid=(N // _RB,),
        in_specs=[
            pl.BlockSpec((_RB, D_IN), lambda i: (i, 0)),
            pl.BlockSpec((D_IN, D_H), lambda i: (0, 0)),
            pl.BlockSpec((2, _RB, 16), lambda i: (0, i, 0)),
        ],
        out_specs=pl.BlockSpec((_RB, D_H), lambda i: (i, 0)),
        out_shape=jax.ShapeDtypeStruct((N, D_H), f32),
    )(x, W1, degp)

    a1 = _agg_kernel(t1.reshape(2 * N, 128), srcs2, dst_f)

    t2 = pl.pallas_call(
        _mm2_body,
        grid=(N // _RB,),
        in_specs=[
            pl.BlockSpec((1, _RB, 128), lambda i: (0, i, 0)),
            pl.BlockSpec((1, _RB, 128), lambda i: (1, i, 0)),
            pl.BlockSpec((D_H, D_H), lambda i: (0, 0)),
            pl.BlockSpec((2, _RB, 16), lambda i: (0, i, 0)),
            pl.BlockSpec((1, D_H), lambda i: (0, 0)),
            pl.BlockSpec((1, D_H), lambda i: (0, 0)),
        ],
        out_specs=pl.BlockSpec((_RB, D_H), lambda i: (i, 0)),
        out_shape=jax.ShapeDtypeStruct((N, D_H), f32),
    )(a1, a1, W2, degp, s1, c1)

    a2 = _agg_kernel(t2.reshape(2 * N, 128), srcs2, dst_f)

    z = pl.pallas_call(
        _mm3_body,
        grid=(N // _RB,),
        in_specs=[
            pl.BlockSpec((1, _RB, 128), lambda i: (0, i, 0)),
            pl.BlockSpec((1, _RB, 128), lambda i: (1, i, 0)),
            pl.BlockSpec((D_H, D_LAT), lambda i: (0, 0)),
            pl.BlockSpec((2, _RB, 16), lambda i: (0, i, 0)),
            pl.BlockSpec((1, D_H), lambda i: (0, 0)),
            pl.BlockSpec((1, D_H), lambda i: (0, 0)),
            pl.BlockSpec((1, D_LAT), lambda i: (0, 0)),
        ],
        out_specs=pl.BlockSpec((_RB, D_LAT), lambda i: (i, 0)),
        out_shape=jax.ShapeDtypeStruct((N, D_LAT), f32),
    )(a2, a2, Wp, degp, s2, c2, bp2)

    return z
